# baseline (speedup 1.0000x reference)
import functools

import jax
import jax.numpy as jnp
from jax import lax
from jax.experimental import pallas as pl
from jax.experimental.pallas import tpu as pltpu
from jax.experimental.pallas import tpu_sc as plsc

_ROW = 128
_K = 10
_NBUF = 2
_BB = 2048


@functools.lru_cache(maxsize=None)
def _make_gather(V, D, B):
    info = plsc.get_sparse_core_info()
    nw = info.num_cores * info.num_subcores
    assert B % (nw * _NBUF * _K * _ROW) == 0
    rows_per_w = B // (nw * _ROW)
    n_pairs = rows_per_w // (_K * _NBUF)
    chunk = _K * _ROW
    mesh = plsc.VectorSubcoreMesh(core_axis_name="c", subcore_axis_name="s")

    @functools.partial(
        pl.kernel,
        mesh=mesh,
        compiler_params=pltpu.CompilerParams(use_tc_tiling_on_sc=False),
        out_type=jax.ShapeDtypeStruct((B, D), jnp.float32),
        scratch_types=[
            pltpu.VMEM((rows_per_w, _ROW), jnp.int32),
            pltpu.VMEM((_NBUF, chunk, D), jnp.float32),
            pltpu.SemaphoreType.DMA,
            pltpu.SemaphoreType.DMA((_NBUF,)),
        ],
    )
    def k(idx_hbm, table_hbm, out_hbm, idx_v, rows_v, gsem, ssem):
        wid = lax.axis_index("s") * info.num_cores + lax.axis_index("c")
        base = wid * rows_per_w
        pltpu.sync_copy(idx_hbm.at[pl.ds(base, rows_per_w)], idx_v)

        def store_desc(b, flat0):
            return pltpu.make_async_copy(
                rows_v.at[b], out_hbm.at[pl.ds(flat0, chunk)], ssem.at[b]
            )

        def pair_body(g, carry):
            for b in range(_NBUF):
                i = g * _NBUF + b
                flat0 = (base + i * _K) * _ROW

                @pl.when(g > 0)
                def _():
                    store_desc(b, flat0).wait()

                copies = [
                    pltpu.async_copy(
                        table_hbm.at[idx_v.at[i * _K + j]],
                        rows_v.at[b].at[pl.ds(j * _ROW, _ROW)],
                        gsem,
                    )
                    for j in range(_K)
                ]
                for c in copies:
                    c.wait()
                store_desc(b, flat0).start()
            return carry

        lax.fori_loop(0, n_pairs, pair_body, 0)
        for b in range(_NBUF):
            store_desc(b, base * _ROW).wait()

    return k


@functools.lru_cache(maxsize=None)
def _make_retile(B, H, D, nh, h0):
    nq = 128 // D
    rb = B * D // 128

    def body(x_ref, *rest):
        o_ref = rest[-1]
        xT = x_ref[0].T
        o_ref[0] = jnp.concatenate(
            [xT[D * q:D * (q + 1)] for q in range(nq)], axis=1
        )

    in_specs = [pl.BlockSpec((1, rb, 128), lambda h: (h, 0, 0))]
    kwargs = {}
    if h0:
        in_specs.append(pl.BlockSpec(memory_space=pl.ANY))
        kwargs = dict(input_output_aliases={1: 0})
    return pl.pallas_call(
        body,
        grid=(nh,),
        in_specs=in_specs,
        out_specs=pl.BlockSpec((1, D, B), lambda h: (h + h0, 0, 0)),
        out_shape=jax.ShapeDtypeStruct((H, D, B), jnp.float32),
        **kwargs,
    )


_RBL = 16384


@functools.lru_cache(maxsize=None)
def _make_table_transpose(V, D):
    nq = 128 // D
    nb = -(-V // (nq * _RBL))

    def body(x_ref, o_ref):
        x = x_ref[...]
        o_ref[...] = jnp.concatenate(
            [x[:, j * _RBL:(j + 1) * _RBL] for j in range(nq)], axis=0
        ).T

    return pl.pallas_call(
        body,
        grid=(nb,),
        in_specs=[pl.BlockSpec((D, nq * _RBL), lambda b: (0, b))],
        out_specs=pl.BlockSpec((_RBL, 128), lambda b: (b, 0)),
        out_shape=jax.ShapeDtypeStruct((nb * _RBL, 128), jnp.float32),
    )


def kernel(input_ids, table):
    B, H = input_ids.shape
    V, D = table.shape
    nq = 128 // D
    tableP = _make_table_transpose(V, D)(table.T).reshape(-1, D)
    ids = input_ids.T.astype(jnp.int32)
    blk = nq * _RBL
    ids = nq * ((ids // blk) * _RBL + (ids % _RBL)) + (ids % blk) // _RBL
    idx = (
        ids.reshape(H, nq, B // nq)
        .transpose(0, 2, 1)
        .reshape(-1, _ROW)
    )
    nh = H // 2
    half_rows = idx.shape[0] // 2
    gath = _make_gather(tableP.shape[0], D, B * H // 2)
    flat_a = gath(idx[:half_rows], tableP)
    flat_b = gath(idx[half_rows:], tableP)
    out_a = _make_retile(B, H, D, nh, 0)(flat_a.reshape(nh, -1, 128))
    outT = _make_retile(B, H, D, nh, nh)(flat_b.reshape(nh, -1, 128), out_a)
    return outT.transpose(2, 0, 1)


# speedup vs baseline: 1.1475x; 1.1475x over previous
import functools

import jax
import jax.numpy as jnp
from jax import lax
from jax.experimental import pallas as pl
from jax.experimental.pallas import tpu as pltpu
from jax.experimental.pallas import tpu_sc as plsc

_ROW = 128
_K = 10
_NBUF = 2
_BB = 2048


@functools.lru_cache(maxsize=None)
def _make_gather(V, D, B):
    info = plsc.get_sparse_core_info()
    nw = info.num_cores * info.num_subcores
    assert B % (nw * _NBUF * _K * _ROW) == 0
    rows_per_w = B // (nw * _ROW)
    n_pairs = rows_per_w // (_K * _NBUF)
    chunk = _K * _ROW
    mesh = plsc.VectorSubcoreMesh(core_axis_name="c", subcore_axis_name="s")

    @functools.partial(
        pl.kernel,
        mesh=mesh,
        compiler_params=pltpu.CompilerParams(use_tc_tiling_on_sc=False),
        out_type=jax.ShapeDtypeStruct((B, D), jnp.float32),
        scratch_types=[
            pltpu.VMEM((rows_per_w, _ROW), jnp.int32),
            pltpu.VMEM((_NBUF, chunk, D), jnp.float32),
            pltpu.SemaphoreType.DMA,
            pltpu.SemaphoreType.DMA((_NBUF,)),
        ],
    )
    def k(idx_hbm, table_hbm, out_hbm, idx_v, rows_v, gsem, ssem):
        wid = lax.axis_index("s") * info.num_cores + lax.axis_index("c")
        base = wid * rows_per_w
        pltpu.sync_copy(idx_hbm.at[pl.ds(base, rows_per_w)], idx_v)

        def store_desc(b, flat0):
            return pltpu.make_async_copy(
                rows_v.at[b], out_hbm.at[pl.ds(flat0, chunk)], ssem.at[b]
            )

        def pair_body(g, carry):
            for b in range(_NBUF):
                i = g * _NBUF + b
                flat0 = (base + i * _K) * _ROW

                @pl.when(g > 0)
                def _():
                    store_desc(b, flat0).wait()

                copies = [
                    pltpu.async_copy(
                        table_hbm.at[idx_v.at[i * _K + j]],
                        rows_v.at[b].at[pl.ds(j * _ROW, _ROW)],
                        gsem,
                    )
                    for j in range(_K)
                ]
                for c in copies:
                    c.wait()
                store_desc(b, flat0).start()
            return carry

        lax.fori_loop(0, n_pairs, pair_body, 0)
        for b in range(_NBUF):
            store_desc(b, base * _ROW).wait()

    return k


@functools.lru_cache(maxsize=None)
def _make_retile(B, H, D):
    nq = 128 // D
    rb = B * D // 128

    def body(x_ref, o_ref):
        for hh in range(2):
            xT = x_ref[hh].T
            o_ref[hh] = jnp.concatenate(
                [xT[D * q:D * (q + 1)] for q in range(nq)], axis=1
            )

    return pl.pallas_call(
        body,
        grid=(H // 2,),
        in_specs=[pl.BlockSpec((2, rb, 128), lambda h: (h, 0, 0))],
        out_specs=pl.BlockSpec((2, D, B), lambda h: (h, 0, 0)),
        out_shape=jax.ShapeDtypeStruct((H, D, B), jnp.float32),
    )


_RBL = 16384


@functools.lru_cache(maxsize=None)
def _make_table_transpose(V, D):
    nq = 128 // D
    nb = -(-V // (nq * _RBL))

    def body(x_ref, o_ref):
        x = x_ref[...]
        o_ref[...] = jnp.concatenate(
            [x[:, j * _RBL:(j + 1) * _RBL] for j in range(nq)], axis=0
        ).T

    return pl.pallas_call(
        body,
        grid=(nb,),
        in_specs=[pl.BlockSpec((D, nq * _RBL), lambda b: (0, b))],
        out_specs=pl.BlockSpec((_RBL, 128), lambda b: (b, 0)),
        out_shape=jax.ShapeDtypeStruct((nb * _RBL, 128), jnp.float32),
    )


def kernel(input_ids, table):
    B, H = input_ids.shape
    V, D = table.shape
    nq = 128 // D
    tableP = _make_table_transpose(V, D)(table.T).reshape(-1, D)
    ids = input_ids.T.astype(jnp.int32)
    blk = nq * _RBL
    ids = nq * ((ids // blk) * _RBL + (ids % _RBL)) + (ids % blk) // _RBL
    idx = (
        ids.reshape(H, nq, B // nq)
        .transpose(0, 2, 1)
        .reshape(-1, _ROW)
    )
    flat = _make_gather(tableP.shape[0], D, B * H)(idx, tableP)
    outT = _make_retile(B, H, D)(flat.reshape(H, -1, 128))
    return outT.transpose(2, 0, 1)


# speedup vs baseline: 1.1536x; 1.0053x over previous
import functools

import jax
import jax.numpy as jnp
from jax import lax
from jax.experimental import pallas as pl
from jax.experimental.pallas import tpu as pltpu
from jax.experimental.pallas import tpu_sc as plsc

_ROW = 128
_K = 10
_NBUF = 2
_BB = 2048


@functools.lru_cache(maxsize=None)
def _make_gather(V, D, B):
    info = plsc.get_sparse_core_info()
    nw = info.num_cores * info.num_subcores
    assert B % (nw * _NBUF * _K * _ROW) == 0
    rows_per_w = B // (nw * _ROW)
    n_pairs = rows_per_w // (_K * _NBUF)
    chunk = _K * _ROW
    mesh = plsc.VectorSubcoreMesh(core_axis_name="c", subcore_axis_name="s")

    @functools.partial(
        pl.kernel,
        mesh=mesh,
        compiler_params=pltpu.CompilerParams(use_tc_tiling_on_sc=False),
        out_type=jax.ShapeDtypeStruct((B, D), jnp.float32),
        scratch_types=[
            pltpu.VMEM((rows_per_w, _ROW), jnp.int32),
            pltpu.VMEM((_NBUF, chunk, D), jnp.float32),
            pltpu.SemaphoreType.DMA,
            pltpu.SemaphoreType.DMA((_NBUF,)),
        ],
    )
    def k(idx_hbm, table_hbm, out_hbm, idx_v, rows_v, gsem, ssem):
        wid = lax.axis_index("s") * info.num_cores + lax.axis_index("c")
        base = wid * rows_per_w
        pltpu.sync_copy(idx_hbm.at[pl.ds(base, rows_per_w)], idx_v)

        def store_desc(b, flat0):
            return pltpu.make_async_copy(
                rows_v.at[b], out_hbm.at[pl.ds(flat0, chunk)], ssem.at[b]
            )

        def pair_body(g, carry):
            for b in range(_NBUF):
                i = g * _NBUF + b
                flat0 = (base + i * _K) * _ROW

                @pl.when(g > 0)
                def _():
                    store_desc(b, flat0).wait()

                copies = [
                    pltpu.async_copy(
                        table_hbm.at[idx_v.at[i * _K + j]],
                        rows_v.at[b].at[pl.ds(j * _ROW, _ROW)],
                        gsem,
                    )
                    for j in range(_K)
                ]
                for c in copies:
                    c.wait()
                store_desc(b, flat0).start()
            return carry

        lax.fori_loop(0, n_pairs, pair_body, 0)
        for b in range(_NBUF):
            store_desc(b, base * _ROW).wait()

    return k


@functools.lru_cache(maxsize=None)
def _make_retile(B, H, D):
    nq = 128 // D
    rb = B * D // 128

    def body(x_ref, o_ref):
        for hh in range(5):
            xT = x_ref[hh].T
            o_ref[hh] = jnp.concatenate(
                [xT[D * q:D * (q + 1)] for q in range(nq)], axis=1
            )

    return pl.pallas_call(
        body,
        grid=(H // 5,),
        in_specs=[pl.BlockSpec((5, rb, 128), lambda h: (h, 0, 0))],
        out_specs=pl.BlockSpec((5, D, B), lambda h: (h, 0, 0)),
        out_shape=jax.ShapeDtypeStruct((H, D, B), jnp.float32),
    )


_RBL = 16384


@functools.lru_cache(maxsize=None)
def _make_table_transpose(V, D):
    nq = 128 // D
    nb = -(-V // (nq * _RBL))

    def body(x_ref, o_ref):
        x = x_ref[...]
        o_ref[...] = jnp.concatenate(
            [x[:, j * _RBL:(j + 1) * _RBL] for j in range(nq)], axis=0
        ).T

    return pl.pallas_call(
        body,
        grid=(nb,),
        in_specs=[pl.BlockSpec((D, nq * _RBL), lambda b: (0, b))],
        out_specs=pl.BlockSpec((_RBL, 128), lambda b: (b, 0)),
        out_shape=jax.ShapeDtypeStruct((nb * _RBL, 128), jnp.float32),
    )


def kernel(input_ids, table):
    B, H = input_ids.shape
    V, D = table.shape
    nq = 128 // D
    tableP = _make_table_transpose(V, D)(table.T).reshape(-1, D)
    ids = input_ids.T.astype(jnp.int32)
    blk = nq * _RBL
    ids = nq * ((ids // blk) * _RBL + (ids % _RBL)) + (ids % blk) // _RBL
    idx = (
        ids.reshape(H, nq, B // nq)
        .transpose(0, 2, 1)
        .reshape(-1, _ROW)
    )
    flat = _make_gather(tableP.shape[0], D, B * H)(idx, tableP)
    outT = _make_retile(B, H, D)(flat.reshape(H, -1, 128))
    return outT.transpose(2, 0, 1)
